# baseline (speedup 1.0000x reference)
import jax
import jax.numpy as jnp
from jax.experimental import pallas as pl
from jax.experimental.pallas import tpu as pltpu

_ROWS_PER_BLOCK = 64
_MAX_ITERS = 16


def _sparsemax_block(x_ref, o_ref):
    m = jnp.max(x_ref[...], axis=-1, keepdims=True)
    tau0 = m - 1.0

    def cond(carry):
        it, tau, prev = carry
        return jnp.logical_and(it < _MAX_ITERS, jnp.any(tau != prev))

    def body(carry):
        it, tau, _ = carry
        s = jnp.sum(jnp.maximum(x_ref[...] - tau, 0.0), axis=-1, keepdims=True)
        c = jnp.sum((x_ref[...] > tau).astype(jnp.float32), axis=-1, keepdims=True)
        new = tau + (s - 1.0) / jnp.maximum(c, 1.0)
        return it + 1, new, tau

    _, tau, _ = jax.lax.while_loop(cond, body, (0, tau0, tau0 - 1.0))
    o_ref[...] = jnp.maximum(x_ref[...] - tau, 0.0)


def kernel(x):
    rows, n = x.shape
    r = _ROWS_PER_BLOCK
    return pl.pallas_call(
        _sparsemax_block,
        out_shape=jax.ShapeDtypeStruct(x.shape, x.dtype),
        grid=(rows // r,),
        in_specs=[pl.BlockSpec((r, n), lambda i: (i, 0))],
        out_specs=pl.BlockSpec((r, n), lambda i: (i, 0)),
        compiler_params=pltpu.CompilerParams(
            dimension_semantics=("parallel",),
        ),
    )(x)
